# baseline (speedup 1.0000x reference)
import functools
import jax
import jax.numpy as jnp
from jax import lax
from jax.experimental import pallas as pl
from jax.experimental.pallas import tpu as pltpu
from jax.experimental.pallas import tpu_sc as plsc

L = 16
UNROLL = 8


def kernel(x):
    n, m = x.shape
    info = plsc.get_sparse_core_info()
    nw = info.num_cores * info.num_subcores
    rows_per_w = n // nw
    mesh = plsc.VectorSubcoreMesh(core_axis_name="c", subcore_axis_name="s")

    @functools.partial(
        pl.kernel,
        mesh=mesh,
        out_type=jax.ShapeDtypeStruct((n, m), jnp.float32),
        scratch_types=[
            pltpu.VMEM((m,), jnp.float32),
            pltpu.VMEM((m,), jnp.float32),
        ],
        compiler_params=pltpu.CompilerParams(needs_layout_passes=False),
    )
    def k(x_hbm, o_hbm, buf_in, buf_out):
        wid = lax.axis_index("s") * info.num_cores + lax.axis_index("c")
        base = wid * rows_per_w

        def row_body(r, _unused):
            row = base + r
            pltpu.sync_copy(x_hbm.at[row], buf_in)

            def chunk(i, carry):
                b = i * (L * UNROLL)
                scans = []
                sums = []
                for u in range(UNROLL):
                    v = buf_in[pl.ds(b + u * L, L)]
                    scans.append(plsc.cumsum(v))
                    sums.append(jnp.sum(v, axis=0))
                c = carry
                for u in range(UNROLL):
                    buf_out[pl.ds(b + u * L, L)] = scans[u] + c
                    c = c + sums[u]
                return c

            lax.fori_loop(0, m // (L * UNROLL), chunk, jnp.float32(0.0))
            pltpu.sync_copy(buf_out, o_hbm.at[row])
            return _unused

        lax.fori_loop(0, rows_per_w, row_body, jnp.int32(0))

    return k(x)


# speedup vs baseline: 2.7325x; 2.7325x over previous
import jax
import jax.numpy as jnp
from jax.experimental import pallas as pl
from jax.experimental.pallas import tpu as pltpu

BR = 512
BC = 4096
LANE = 128


def _tri(n, dtype):
    r = jax.lax.broadcasted_iota(jnp.int32, (n, n), 0)
    c = jax.lax.broadcasted_iota(jnp.int32, (n, n), 1)
    return (r <= c).astype(dtype)


def _scan_kernel(x_ref, o_ref, carry_ref):
    j = pl.program_id(1)

    @pl.when(j == 0)
    def _():
        carry_ref[...] = jnp.zeros_like(carry_ref)

    t = _tri(LANE, jnp.float32)
    run = carry_ref[:, :1]
    for k in range(BC // LANE):
        sl = slice(k * LANE, (k + 1) * LANE)
        y = jax.lax.dot_general(
            x_ref[:, sl], t, (((1,), (0,)), ((), ())),
            preferred_element_type=jnp.float32,
        )
        y = y + run
        o_ref[:, sl] = y
        run = y[:, LANE - 1 :]
    carry_ref[...] = jnp.broadcast_to(run, carry_ref.shape)


def kernel(x):
    n, m = x.shape
    grid = (n // BR, m // BC)
    return pl.pallas_call(
        _scan_kernel,
        grid=grid,
        in_specs=[pl.BlockSpec((BR, BC), lambda i, j: (i, j))],
        out_specs=pl.BlockSpec((BR, BC), lambda i, j: (i, j)),
        out_shape=jax.ShapeDtypeStruct((n, m), x.dtype),
        scratch_shapes=[pltpu.VMEM((BR, LANE), jnp.float32)],
        compiler_params=pltpu.CompilerParams(
            dimension_semantics=("parallel", "arbitrary"),
        ),
    )(x)
